# Initial kernel scaffold; baseline (speedup 1.0000x reference)
import jax
import jax.numpy as jnp
from jax.experimental import pallas as pl


def kernel(fuzzified_x, input_selectors):
    raise NotImplementedError("write your pallas kernel here")


# speedup vs baseline: 40445.7810x; 40445.7810x over previous
import jax
import jax.numpy as jnp
from jax.experimental import pallas as pl

_N_MEM = 16


def _fuzzy_body(x_ref, sel_ref, out_ref):
    n_inputs, n_rules = sel_ref.shape
    sel = jnp.round(sel_ref[...] * _N_MEM).astype(jnp.int32)
    m_iota = jax.lax.broadcasted_iota(
        jnp.int32, (n_inputs, _N_MEM, n_rules), 1)
    onehot = (sel[:, None, :] == m_iota).astype(jnp.float32)
    onehot = onehot.reshape(n_inputs * _N_MEM, n_rules)
    logx = jnp.maximum(jnp.log(x_ref[...]), jnp.float32(-1e5))
    acc = jax.lax.dot_general(
        logx, onehot, (((1,), (0,)), ((), ())),
        preferred_element_type=jnp.float32,
        precision=jax.lax.Precision.HIGHEST)
    out_ref[...] = jnp.exp(acc)


def kernel(fuzzified_x, input_selectors):
    b, n_inputs, n_mem = fuzzified_x.shape
    n_rules = input_selectors.shape[1]
    x2 = fuzzified_x.reshape(b, n_inputs * n_mem)
    bt = 256
    return pl.pallas_call(
        _fuzzy_body,
        grid=(b // bt,),
        in_specs=[
            pl.BlockSpec((bt, n_inputs * n_mem), lambda i: (i, 0)),
            pl.BlockSpec((n_inputs, n_rules), lambda i: (0, 0)),
        ],
        out_specs=pl.BlockSpec((bt, n_rules), lambda i: (i, 0)),
        out_shape=jax.ShapeDtypeStruct((b, n_rules), jnp.float32),
    )(x2, input_selectors)
